# baseline (speedup 1.0000x reference)
import functools

import jax
import jax.numpy as jnp
from jax import lax
from jax.experimental import pallas as pl
from jax.experimental.pallas import tpu as pltpu
from jax.experimental.pallas import tpu_sc as plsc

NUM_CORES = 2
NUM_SUBCORES = 16
LANES = 16
NUM_WORKERS = NUM_CORES * NUM_SUBCORES

NUM_CB = 4
CB_K = 512
DIM = 64
NTOK = 1024
TOK_PER_W = NTOK // NUM_WORKERS
TBLK = TOK_PER_W // LANES
ZW = DIM * TOK_PER_W
IW = NUM_CB * TOK_PER_W


def _tree_sum(leaves):
    parts = []
    for j in range(8):
        x = leaves[8 * j : 8 * j + 8]
        s1 = [x[i] + x[i + 4] for i in range(4)]
        s2 = [s1[0] + s1[2], s1[1] + s1[3]]
        parts.append(s2[0] + s2[1])
    l1 = [parts[0] + parts[1], parts[2] + parts[3],
          parts[4] + parts[5], parts[6] + parts[7]]
    return (l1[0] + l1[1]) + (l1[2] + l1[3])


_mesh = plsc.VectorSubcoreMesh(core_axis_name="c", subcore_axis_name="s", num_cores=NUM_CORES)


@functools.partial(
    pl.kernel,
    out_type=(
        jax.ShapeDtypeStruct((NUM_WORKERS * ZW,), jnp.float32),
        jax.ShapeDtypeStruct((NUM_WORKERS * IW,), jnp.int32),
    ),
    mesh=_mesh,
    compiler_params=pltpu.CompilerParams(needs_layout_passes=False),
    scratch_types=[
        pltpu.VMEM((ZW,), jnp.float32),
        pltpu.VMEM((ZW,), jnp.float32),
        pltpu.VMEM((CB_K * DIM,), jnp.float32),
        pltpu.VMEM((IW,), jnp.int32),
    ],
)
def _rvq_sc(z_hbm, cb_hbm, zq_hbm, idx_hbm, rT, qT, cbv, idxv):
    wid = lax.axis_index("s") * NUM_CORES + lax.axis_index("c")
    zsl = pl.ds(wid * ZW, ZW)
    pltpu.sync_copy(z_hbm.at[zsl], rT)

    iota = lax.broadcasted_iota(jnp.int32, (LANES,), 0)

    for stage in range(NUM_CB):
        pltpu.sync_copy(cb_hbm.at[stage], cbv)

        for tb in range(TBLK):
            def code_body(c, carry, tb=tb):
                runmin, runidx = carry
                rows = [cbv[pl.ds(c * DIM + 16 * jj, 16)] for jj in range(4)]
                leaves = []
                for d in range(DIM):
                    cbs = rows[d // 16][d % 16]
                    diff = rT[pl.ds(d * TOK_PER_W + tb * LANES, LANES)] - cbs
                    leaves.append(diff * diff)
                dist = _tree_sum(leaves)
                better = dist < runmin
                runidx = jnp.where(better, c, runidx)
                runmin = jnp.where(better, dist, runmin)
                return runmin, runidx

            runmin0 = jnp.full((LANES,), jnp.inf, jnp.float32)
            runidx0 = jnp.zeros((LANES,), jnp.int32)
            _, runidx = lax.fori_loop(0, CB_K, code_body, (runmin0, runidx0))

            idxv[pl.ds(stage * TOK_PER_W + tb * LANES, LANES)] = runidx
            tok = iota + tb * LANES

            def upd_body(d, _, runidx=runidx, tok=tok, first=(stage == 0)):
                dvec = jnp.full((LANES,), d, jnp.int32)
                flat = dvec * TOK_PER_W + tok
                qd = plsc.load_gather(cbv, [runidx * DIM + dvec])
                rd = plsc.load_gather(rT, [flat])
                q_used = rd + (qd - rd)
                plsc.store_scatter(rT, [flat], rd - q_used)
                if first:
                    plsc.store_scatter(qT, [flat], q_used)
                else:
                    qacc = plsc.load_gather(qT, [flat])
                    plsc.store_scatter(qT, [flat], qacc + q_used)
                return 0

            lax.fori_loop(0, DIM, upd_body, 0)

    pltpu.sync_copy(z_hbm.at[zsl], rT)
    for tb in range(TBLK):
        tok = iota + tb * LANES

        def fin_body(d, _, tok=tok):
            flat = jnp.full((LANES,), d, jnp.int32) * TOK_PER_W + tok
            zd = plsc.load_gather(rT, [flat])
            qs = plsc.load_gather(qT, [flat])
            plsc.store_scatter(qT, [flat], zd + (qs - zd))
            return 0

        lax.fori_loop(0, DIM, fin_body, 0)

    pltpu.sync_copy(qT, zq_hbm.at[zsl])
    pltpu.sync_copy(idxv, idx_hbm.at[pl.ds(wid * IW, IW)])


def kernel(z, codebooks):
    B, T, D = z.shape
    zw = (z.reshape(NUM_WORKERS, TOK_PER_W, D)
          .transpose(0, 2, 1)
          .reshape(NUM_WORKERS * ZW))
    zq_flat, idx_flat = _rvq_sc(zw, codebooks.reshape(NUM_CB, CB_K * DIM))
    z_q_final = (zq_flat.reshape(NUM_WORKERS, D, TOK_PER_W)
                 .transpose(0, 2, 1)
                 .reshape(B, T, D))
    all_indices = (idx_flat.reshape(NUM_WORKERS, NUM_CB, TOK_PER_W)
                   .transpose(0, 2, 1)
                   .reshape(B, T, NUM_CB))
    return (z_q_final, all_indices)


# speedup vs baseline: 1.3885x; 1.3885x over previous
import functools

import jax
import jax.numpy as jnp
from jax import lax
from jax.experimental import pallas as pl
from jax.experimental.pallas import tpu as pltpu
from jax.experimental.pallas import tpu_sc as plsc

NUM_CORES = 2
NUM_SUBCORES = 16
LANES = 16
NUM_WORKERS = NUM_CORES * NUM_SUBCORES

NUM_CB = 4
CB_K = 512
DIM = 64
NTOK = 1024
TOK_PER_W = NTOK // NUM_WORKERS
TBLK = TOK_PER_W // LANES
ZW = DIM * TOK_PER_W
IW = NUM_CB * TOK_PER_W
PBUF = CB_K * LANES

_mesh = plsc.VectorSubcoreMesh(core_axis_name="c", subcore_axis_name="s",
                               num_cores=NUM_CORES)


@functools.partial(
    pl.kernel,
    out_type=(
        jax.ShapeDtypeStruct((NUM_WORKERS * ZW,), jnp.float32),
        jax.ShapeDtypeStruct((NUM_WORKERS * IW,), jnp.int32),
    ),
    mesh=_mesh,
    compiler_params=pltpu.CompilerParams(needs_layout_passes=False),
    scratch_types=[
        pltpu.VMEM((ZW,), jnp.float32),
        pltpu.VMEM((ZW,), jnp.float32),
        pltpu.VMEM((CB_K * DIM,), jnp.float32),
        pltpu.VMEM((IW,), jnp.int32),
        pltpu.VMEM((TBLK * PBUF,), jnp.float32),
        pltpu.VMEM((TBLK * PBUF,), jnp.float32),
    ],
)
def _rvq_sc(z_hbm, cb_hbm, zq_hbm, idx_hbm, rT, qT, cbv, idxv, pA, pB):
    wid = lax.axis_index("s") * NUM_CORES + lax.axis_index("c")
    zsl = pl.ds(wid * ZW, ZW)
    pltpu.sync_copy(z_hbm.at[zsl], rT)

    iota = lax.broadcasted_iota(jnp.int32, (LANES,), 0)

    for stage in range(NUM_CB):
        pltpu.sync_copy(cb_hbm.at[stage], cbv)

        def make_pass(p, carried):
            def pass_body(c, carry):
                row = cbv[pl.ds(c * DIM + 16 * p, 16)]
                ts = []
                for tb in range(TBLK):
                    x = []
                    for k in range(16):
                        d = 16 * p + k
                        cbs = row[k]
                        xv = rT[pl.ds(d * TOK_PER_W + tb * LANES, LANES)] - cbs
                        x.append(xv * xv)
                    ps = []
                    for blk in range(2):
                        y = x[8 * blk: 8 * blk + 8]
                        s1 = [y[i] + y[i + 4] for i in range(4)]
                        ps.append((s1[0] + s1[2]) + (s1[1] + s1[3]))
                    ts.append(ps[0] + ps[1])

                if p == 0:
                    for tb in range(TBLK):
                        pA[pl.ds(tb * PBUF + c * LANES, LANES)] = ts[tb]
                    return carry
                if p == 1:
                    for tb in range(TBLK):
                        sl = pl.ds(tb * PBUF + c * LANES, LANES)
                        pA[sl] = pA[sl] + ts[tb]
                    return carry
                if p == 2:
                    for tb in range(TBLK):
                        pB[pl.ds(tb * PBUF + c * LANES, LANES)] = ts[tb]
                    return carry
                rm0, ri0, rm1, ri1 = carry
                sl0 = pl.ds(c * LANES, LANES)
                sl1 = pl.ds(PBUF + c * LANES, LANES)
                dd0 = pA[sl0] + (pB[sl0] + ts[0])
                dd1 = pA[sl1] + (pB[sl1] + ts[1])
                b0 = dd0 < rm0
                b1 = dd1 < rm1
                ri0 = jnp.where(b0, c, ri0)
                rm0 = jnp.where(b0, dd0, rm0)
                ri1 = jnp.where(b1, c, ri1)
                rm1 = jnp.where(b1, dd1, rm1)
                return rm0, ri0, rm1, ri1

            return lax.fori_loop(0, CB_K, pass_body, carried)

        for p in range(3):
            make_pass(p, 0)
        inf0 = jnp.full((LANES,), jnp.inf, jnp.float32)
        zi0 = jnp.zeros((LANES,), jnp.int32)
        _, ridx0, _, ridx1 = make_pass(3, (inf0, zi0, inf0, zi0))

        for tb in range(TBLK):
            runidx = (ridx0, ridx1)[tb]
            idxv[pl.ds(stage * TOK_PER_W + tb * LANES, LANES)] = runidx
            tok = iota + tb * LANES

            def upd_body(d, _, runidx=runidx, tok=tok, first=(stage == 0)):
                dvec = jnp.full((LANES,), d, jnp.int32)
                flat = dvec * TOK_PER_W + tok
                qd = plsc.load_gather(cbv, [runidx * DIM + dvec])
                rd = plsc.load_gather(rT, [flat])
                q_used = rd + (qd - rd)
                plsc.store_scatter(rT, [flat], rd - q_used)
                if first:
                    plsc.store_scatter(qT, [flat], q_used)
                else:
                    qacc = plsc.load_gather(qT, [flat])
                    plsc.store_scatter(qT, [flat], qacc + q_used)
                return 0

            lax.fori_loop(0, DIM, upd_body, 0)

    pltpu.sync_copy(z_hbm.at[zsl], rT)
    for tb in range(TBLK):
        tok = iota + tb * LANES

        def fin_body(d, _, tok=tok):
            flat = jnp.full((LANES,), d, jnp.int32) * TOK_PER_W + tok
            zd = plsc.load_gather(rT, [flat])
            qs = plsc.load_gather(qT, [flat])
            plsc.store_scatter(qT, [flat], zd + (qs - zd))
            return 0

        lax.fori_loop(0, DIM, fin_body, 0)

    pltpu.sync_copy(qT, zq_hbm.at[zsl])
    pltpu.sync_copy(idxv, idx_hbm.at[pl.ds(wid * IW, IW)])


def kernel(z, codebooks):
    B, T, D = z.shape
    zw = (z.reshape(NUM_WORKERS, TOK_PER_W, D)
          .transpose(0, 2, 1)
          .reshape(NUM_WORKERS * ZW))
    zq_flat, idx_flat = _rvq_sc(zw, codebooks.reshape(NUM_CB, CB_K * DIM))
    z_q_final = (zq_flat.reshape(NUM_WORKERS, D, TOK_PER_W)
                 .transpose(0, 2, 1)
                 .reshape(B, T, D))
    all_indices = (idx_flat.reshape(NUM_WORKERS, NUM_CB, TOK_PER_W)
                   .transpose(0, 2, 1)
                   .reshape(B, T, NUM_CB))
    return (z_q_final, all_indices)


# speedup vs baseline: 1.5588x; 1.1227x over previous
import functools

import jax
import jax.numpy as jnp
from jax import lax
from jax.experimental import pallas as pl
from jax.experimental.pallas import tpu as pltpu
from jax.experimental.pallas import tpu_sc as plsc

NUM_CORES = 2
NUM_SUBCORES = 16
LANES = 16
NUM_WORKERS = NUM_CORES * NUM_SUBCORES

NUM_CB = 4
CB_K = 512
DIM = 64
NTOK = 1024
TOK_PER_W = NTOK // NUM_WORKERS
TBLK = TOK_PER_W // LANES
ZW = DIM * TOK_PER_W
IW = NUM_CB * TOK_PER_W
PBUF = CB_K * LANES

_mesh = plsc.VectorSubcoreMesh(core_axis_name="c", subcore_axis_name="s",
                               num_cores=NUM_CORES)


@functools.partial(
    pl.kernel,
    out_type=(
        jax.ShapeDtypeStruct((NUM_WORKERS * ZW,), jnp.float32),
        jax.ShapeDtypeStruct((NUM_WORKERS * IW,), jnp.int32),
    ),
    mesh=_mesh,
    compiler_params=pltpu.CompilerParams(needs_layout_passes=False),
    scratch_types=[
        pltpu.VMEM((ZW,), jnp.float32),
        pltpu.VMEM((ZW,), jnp.float32),
        pltpu.VMEM((CB_K * DIM,), jnp.float32),
        pltpu.VMEM((CB_K * DIM,), jnp.float32),
        pltpu.VMEM((IW,), jnp.int32),
        pltpu.VMEM((TBLK * PBUF,), jnp.float32),
        pltpu.VMEM((TBLK * PBUF,), jnp.float32),
        pltpu.SemaphoreType.DMA,
        pltpu.SemaphoreType.DMA,
    ],
)
def _rvq_sc(z_hbm, cb_hbm, zq_hbm, idx_hbm, rT, qT, cb0, cb1, idxv, pA, pB,
            sem0, sem1):
    wid = lax.axis_index("s") * NUM_CORES + lax.axis_index("c")
    zsl = pl.ds(wid * ZW, ZW)
    cbufs = (cb0, cb1)
    sems = (sem0, sem1)
    pending = pltpu.async_copy(cb_hbm.at[0], cb0, sem0)
    pltpu.sync_copy(z_hbm.at[zsl], rT)

    iota = lax.broadcasted_iota(jnp.int32, (LANES,), 0)

    for stage in range(NUM_CB):
        cbv = cbufs[stage % 2]
        pending.wait()
        if stage + 1 < NUM_CB:
            pending = pltpu.async_copy(
                cb_hbm.at[stage + 1], cbufs[(stage + 1) % 2],
                sems[(stage + 1) % 2])

        def make_pass(p, carried):
            rvecs = tuple(
                rT[pl.ds((16 * p + k) * TOK_PER_W + tb * LANES, LANES)]
                for tb in range(TBLK) for k in range(16))

            def pass_body(c, carry):
                rv = carry[:2 * LANES]
                rest = carry[2 * LANES:]
                row = cbv[pl.ds(c * DIM + 16 * p, 16)]
                ts = []
                for tb in range(TBLK):
                    x = []
                    for k in range(16):
                        cbs = row[k]
                        xv = rv[tb * 16 + k] - cbs
                        x.append(xv * xv)
                    ps = []
                    for blk in range(2):
                        y = x[8 * blk: 8 * blk + 8]
                        s1 = [y[i] + y[i + 4] for i in range(4)]
                        ps.append((s1[0] + s1[2]) + (s1[1] + s1[3]))
                    ts.append(ps[0] + ps[1])

                if p == 0:
                    for tb in range(TBLK):
                        pA[pl.ds(tb * PBUF + c * LANES, LANES)] = ts[tb]
                    return rv + rest
                if p == 1:
                    for tb in range(TBLK):
                        sl = pl.ds(tb * PBUF + c * LANES, LANES)
                        pA[sl] = pA[sl] + ts[tb]
                    return rv + rest
                if p == 2:
                    for tb in range(TBLK):
                        pB[pl.ds(tb * PBUF + c * LANES, LANES)] = ts[tb]
                    return rv + rest
                rm0, ri0, rm1, ri1 = rest
                sl0 = pl.ds(c * LANES, LANES)
                sl1 = pl.ds(PBUF + c * LANES, LANES)
                dd0 = pA[sl0] + (pB[sl0] + ts[0])
                dd1 = pA[sl1] + (pB[sl1] + ts[1])
                b0 = dd0 < rm0
                b1 = dd1 < rm1
                ri0 = jnp.where(b0, c, ri0)
                rm0 = jnp.where(b0, dd0, rm0)
                ri1 = jnp.where(b1, c, ri1)
                rm1 = jnp.where(b1, dd1, rm1)
                return rv + (rm0, ri0, rm1, ri1)

            out = lax.fori_loop(0, CB_K, pass_body, rvecs + tuple(carried),
                                unroll=1)
            return out[2 * LANES:]

        for p in range(3):
            make_pass(p, ())
        inf0 = jnp.full((LANES,), jnp.inf, jnp.float32)
        zi0 = jnp.zeros((LANES,), jnp.int32)
        _, ridx0, _, ridx1 = make_pass(3, (inf0, zi0, inf0, zi0))

        for tb in range(TBLK):
            runidx = (ridx0, ridx1)[tb]
            idxv[pl.ds(stage * TOK_PER_W + tb * LANES, LANES)] = runidx
            tok = iota + tb * LANES

            def upd_body(d, _, runidx=runidx, tok=tok, first=(stage == 0)):
                dvec = jnp.full((LANES,), d, jnp.int32)
                flat = dvec * TOK_PER_W + tok
                qd = plsc.load_gather(cbv, [runidx * DIM + dvec])
                rd = plsc.load_gather(rT, [flat])
                q_used = rd + (qd - rd)
                plsc.store_scatter(rT, [flat], rd - q_used)
                if first:
                    plsc.store_scatter(qT, [flat], q_used)
                else:
                    qacc = plsc.load_gather(qT, [flat])
                    plsc.store_scatter(qT, [flat], qacc + q_used)
                return 0

            lax.fori_loop(0, DIM, upd_body, 0)

    pltpu.sync_copy(z_hbm.at[zsl], rT)
    for tb in range(TBLK):
        tok = iota + tb * LANES

        def fin_body(d, _, tok=tok):
            flat = jnp.full((LANES,), d, jnp.int32) * TOK_PER_W + tok
            zd = plsc.load_gather(rT, [flat])
            qs = plsc.load_gather(qT, [flat])
            plsc.store_scatter(qT, [flat], zd + (qs - zd))
            return 0

        lax.fori_loop(0, DIM, fin_body, 0)

    pltpu.sync_copy(qT, zq_hbm.at[zsl])
    pltpu.sync_copy(idxv, idx_hbm.at[pl.ds(wid * IW, IW)])


def kernel(z, codebooks):
    B, T, D = z.shape
    zw = (z.reshape(NUM_WORKERS, TOK_PER_W, D)
          .transpose(0, 2, 1)
          .reshape(NUM_WORKERS * ZW))
    zq_flat, idx_flat = _rvq_sc(zw, codebooks.reshape(NUM_CB, CB_K * DIM))
    z_q_final = (zq_flat.reshape(NUM_WORKERS, D, TOK_PER_W)
                 .transpose(0, 2, 1)
                 .reshape(B, T, D))
    all_indices = (idx_flat.reshape(NUM_WORKERS, NUM_CB, TOK_PER_W)
                   .transpose(0, 2, 1)
                   .reshape(B, T, NUM_CB))
    return (z_q_final, all_indices)
